# baseline (speedup 1.0000x reference)
import jax
import jax.numpy as jnp
from jax import lax
from jax.experimental import pallas as pl
from jax.experimental.pallas import tpu as pltpu
from jax.experimental.pallas import tpu_sc as plsc

N = 10000
E = 320000
F = 128

NC = 2
NS = 16
FH = F // NC
E_PER = E // NS
CH = 80
NCH = E_PER // CH
ROWS_PER_TILE = N // NS


def _tc_pre_body(x_ref, w_ref, a1_ref, a2_ref, h_ref, f1_ref, f2_ref):
    h = jnp.dot(x_ref[...], w_ref[...], preferred_element_type=jnp.float32)
    h_ref[...] = h
    f1_ref[...] = jnp.dot(h, a1_ref[...], preferred_element_type=jnp.float32)
    f2_ref[...] = jnp.dot(h, a2_ref[...], preferred_element_type=jnp.float32)


_tc_pre = pl.pallas_call(
    _tc_pre_body,
    out_shape=[
        jax.ShapeDtypeStruct((N, F), jnp.float32),
        jax.ShapeDtypeStruct((N, 1), jnp.float32),
        jax.ShapeDtypeStruct((N, 1), jnp.float32),
    ],
)


def _sc_body(src2_hbm, dst2_hbm, f1_hbm, f2_hbm, h2_hbm, zeros_hbm,
             ee_hbm, part_hbm,
             src2_v, dst2_v, w2_v, f1_v, f2_v, rows_a, rows_b, shared,
             gsem_a, gsem_b, ssem_a, ssem_b):
    c = lax.axis_index("c")
    s = lax.axis_index("s")

    pltpu.sync_copy(src2_hbm.at[s], src2_v)
    pltpu.sync_copy(dst2_hbm.at[s], dst2_v)
    pltpu.sync_copy(f1_hbm, f1_v)
    pltpu.sync_copy(f2_hbm, f2_v)

    pltpu.sync_copy(zeros_hbm.at[s],
                    shared.at[pl.ds(s * ROWS_PER_TILE, ROWS_PER_TILE)])

    @plsc.parallel_loop(0, NCH, unroll=2)
    def wbody(ci):
        for k in range(CH // 16):
            si = src2_v[ci, pl.ds(k * 16, 16)]
            di = dst2_v[ci, pl.ds(k * 16, 16)]
            sc = plsc.load_gather(f1_v, [si]) + plsc.load_gather(f2_v, [di])
            lr = jnp.where(sc >= 0.0, sc, sc * 0.2)
            w2_v[ci, pl.ds(k * 16, 16)] = jnp.exp(-lr)

    @pl.when(c == 0)
    def _():
        pltpu.sync_copy(w2_v, ee_hbm.at[s])

    plsc.subcore_barrier()

    hview = h2_hbm.at[c]

    def scale(rows, ci):
        civ = jnp.full((16,), ci, jnp.int32)

        @plsc.parallel_loop(0, CH, unroll=16)
        def ebody(k):
            wv = plsc.load_gather(
                w2_v, [civ, jnp.full((16,), k, jnp.int32)])
            for j in range(FH // 16):
                rows[k, pl.ds(j * 16, 16)] = rows[k, pl.ds(j * 16, 16)] * wv

    pltpu.async_copy(hview.at[dst2_v.at[0]], rows_a, gsem_a)

    def mbody(cio, _):
        ci = cio * 2
        pltpu.async_copy(hview.at[dst2_v.at[ci + 1]], rows_b, gsem_b)
        pltpu.make_async_copy(hview.at[dst2_v.at[ci]], rows_a, gsem_a).wait()
        scale(rows_a, ci)
        pltpu.async_copy(rows_a, shared.at[src2_v.at[ci]], ssem_a, add=True)
        pltpu.make_async_copy(hview.at[dst2_v.at[ci + 1]], rows_b,
                              gsem_b).wait()
        scale(rows_b, ci + 1)
        pltpu.async_copy(rows_b, shared.at[src2_v.at[ci + 1]], ssem_b,
                         add=True)
        pltpu.make_async_copy(rows_a, shared.at[src2_v.at[ci]], ssem_a).wait()

        @pl.when(ci + 2 < NCH)
        def _():
            pltpu.async_copy(hview.at[dst2_v.at[ci + 2]], rows_a, gsem_a)

        pltpu.make_async_copy(rows_b, shared.at[src2_v.at[ci + 1]],
                              ssem_b).wait()
        return 0

    lax.fori_loop(0, NCH // 2, mbody, 0)
    plsc.subcore_barrier()

    pltpu.sync_copy(shared.at[pl.ds(s * ROWS_PER_TILE, ROWS_PER_TILE)],
                    part_hbm.at[c, s])


_sc_edge = pl.kernel(
    _sc_body,
    out_type=[
        jax.ShapeDtypeStruct((NS, NCH, CH), jnp.float32),
        jax.ShapeDtypeStruct((NC, NS, ROWS_PER_TILE, FH), jnp.float32),
    ],
    mesh=plsc.VectorSubcoreMesh(core_axis_name="c", subcore_axis_name="s"),
    compiler_params=pltpu.CompilerParams(
        needs_layout_passes=False, use_tc_tiling_on_sc=False),
    scratch_types=[
        pltpu.VMEM((NCH, CH), jnp.int32),
        pltpu.VMEM((NCH, CH), jnp.int32),
        pltpu.VMEM((NCH, CH), jnp.float32),
        pltpu.VMEM((N,), jnp.float32),
        pltpu.VMEM((N,), jnp.float32),
        pltpu.VMEM((CH, FH), jnp.float32),
        pltpu.VMEM((CH, FH), jnp.float32),
        pltpu.VMEM_SHARED((N, FH), jnp.float32),
        pltpu.SemaphoreType.DMA,
        pltpu.SemaphoreType.DMA,
        pltpu.SemaphoreType.DMA,
        pltpu.SemaphoreType.DMA,
    ],
)


def _tc_post_body(p0_ref, p1_ref, o_ref):
    x = jnp.concatenate([p0_ref[...], p1_ref[...]], axis=1)
    o_ref[...] = jnp.where(x > 0.0, x, jnp.exp(x) - 1.0)


_tc_post = pl.pallas_call(
    _tc_post_body,
    out_shape=jax.ShapeDtypeStruct((N, F), jnp.float32),
)


def kernel(non_zero, input, W, a):
    src = non_zero[0, :]
    dst = non_zero[1, :]
    a1 = a[0, :F].reshape(F, 1)
    a2 = a[0, F:].reshape(F, 1)
    h, f1, f2 = _tc_pre(input, W, a1, a2)
    h2 = jnp.stack([h[:, :FH], h[:, FH:]])
    src2 = src.reshape(NS, NCH, CH)
    dst2 = dst.reshape(NS, NCH, CH)
    zeros = jnp.zeros((NS, ROWS_PER_TILE, FH), jnp.float32)
    ee, part = _sc_edge(src2, dst2, f1.reshape(N), f2.reshape(N), h2, zeros)
    out = _tc_post(part[0].reshape(N, FH), part[1].reshape(N, FH))
    return out, ee.reshape(E)


# speedup vs baseline: 1.2426x; 1.2426x over previous
import jax
import jax.numpy as jnp
from jax import lax
from jax.experimental import pallas as pl
from jax.experimental.pallas import tpu as pltpu
from jax.experimental.pallas import tpu_sc as plsc

N = 10000
E = 320000
F = 128

NC = 2
NS = 16
FH = F // NC
E_PER = E // NS
CH = 80
NCH = E_PER // CH
ROWS_PER_TILE = N // NS


def _tc_pre_body(x_ref, w_ref, a12_ref, h_ref, f12_ref):
    h = jnp.dot(x_ref[...], w_ref[...], preferred_element_type=jnp.float32)
    h_ref[...] = h
    f12_ref[...] = lax.dot_general(
        a12_ref[...], h, (((1,), (1,)), ((), ())),
        preferred_element_type=jnp.float32)


_tc_pre = pl.pallas_call(
    _tc_pre_body,
    out_shape=[
        jax.ShapeDtypeStruct((N, F), jnp.float32),
        jax.ShapeDtypeStruct((NC, N), jnp.float32),
    ],
)


def _sc_body(src2_hbm, dst2_hbm, f12_hbm, hflat_hbm, zeros_hbm,
             ee_hbm, hp_hbm,
             src2_v, dst2_v, w2_v, f1_v, f2_v, rows_a, rows_b, shared,
             gsem_a, gsem_b, ssem_a, ssem_b):
    c = lax.axis_index("c")
    s = lax.axis_index("s")

    pltpu.sync_copy(src2_hbm.at[s], src2_v)
    pltpu.sync_copy(dst2_hbm.at[s], dst2_v)
    pltpu.sync_copy(f12_hbm.at[0], f1_v)
    pltpu.sync_copy(f12_hbm.at[1], f2_v)

    pltpu.sync_copy(zeros_hbm.at[s],
                    shared.at[pl.ds(s * ROWS_PER_TILE, ROWS_PER_TILE)])

    @plsc.parallel_loop(0, NCH, unroll=2)
    def wbody(ci):
        for k in range(CH // 16):
            si = src2_v[ci, pl.ds(k * 16, 16)]
            di = dst2_v[ci, pl.ds(k * 16, 16)]
            sc = plsc.load_gather(f1_v, [si]) + plsc.load_gather(f2_v, [di])
            lr = jnp.where(sc >= 0.0, sc, sc * 0.2)
            w2_v[ci, pl.ds(k * 16, 16)] = jnp.exp(-lr)
            dst2_v[ci, pl.ds(k * 16, 16)] = di * 2 + c

    @pl.when(c == 0)
    def _():
        pltpu.sync_copy(w2_v, ee_hbm.at[s])

    plsc.subcore_barrier()

    def scale(rows, ci):
        civ = jnp.full((16,), ci, jnp.int32)

        @plsc.parallel_loop(0, CH, unroll=8)
        def ebody(k):
            wv = plsc.load_gather(
                w2_v, [civ, jnp.full((16,), k, jnp.int32)])
            for j in range(FH // 16):
                rows[k, pl.ds(j * 16, 16)] = rows[k, pl.ds(j * 16, 16)] * wv

    hview = hflat_hbm
    pltpu.async_copy(hview.at[dst2_v.at[0]], rows_a, gsem_a)

    def mbody(cio, _):
        ci = cio * 2
        pltpu.async_copy(hview.at[dst2_v.at[ci + 1]], rows_b, gsem_b)
        pltpu.make_async_copy(hview.at[dst2_v.at[ci]], rows_a,
                              gsem_a).wait()
        scale(rows_a, ci)
        pltpu.async_copy(rows_a, shared.at[src2_v.at[ci]], ssem_a, add=True)
        pltpu.make_async_copy(hview.at[dst2_v.at[ci + 1]], rows_b,
                              gsem_b).wait()
        scale(rows_b, ci + 1)
        pltpu.async_copy(rows_b, shared.at[src2_v.at[ci + 1]], ssem_b,
                         add=True)
        pltpu.make_async_copy(rows_a, shared.at[src2_v.at[ci]], ssem_a).wait()

        @pl.when(ci + 2 < NCH)
        def _():
            pltpu.async_copy(hview.at[dst2_v.at[ci + 2]], rows_a, gsem_a)

        pltpu.make_async_copy(rows_b, shared.at[src2_v.at[ci + 1]],
                              ssem_b).wait()
        return 0

    lax.fori_loop(0, NCH // 2, mbody, 0)
    plsc.subcore_barrier()

    pltpu.sync_copy(shared.at[pl.ds(s * ROWS_PER_TILE, ROWS_PER_TILE)],
                    hp_hbm.at[pl.ds(s * ROWS_PER_TILE, ROWS_PER_TILE),
                              pl.ds(c * FH, FH)])


_sc_edge = pl.kernel(
    _sc_body,
    out_type=[
        jax.ShapeDtypeStruct((NS, NCH, CH), jnp.float32),
        jax.ShapeDtypeStruct((N, F), jnp.float32),
    ],
    mesh=plsc.VectorSubcoreMesh(core_axis_name="c", subcore_axis_name="s"),
    compiler_params=pltpu.CompilerParams(
        needs_layout_passes=False, use_tc_tiling_on_sc=False),
    scratch_types=[
        pltpu.VMEM((NCH, CH), jnp.int32),
        pltpu.VMEM((NCH, CH), jnp.int32),
        pltpu.VMEM((NCH, CH), jnp.float32),
        pltpu.VMEM((N,), jnp.float32),
        pltpu.VMEM((N,), jnp.float32),
        pltpu.VMEM((CH, FH), jnp.float32),
        pltpu.VMEM((CH, FH), jnp.float32),
        pltpu.VMEM_SHARED((N, FH), jnp.float32),
        pltpu.SemaphoreType.DMA,
        pltpu.SemaphoreType.DMA,
        pltpu.SemaphoreType.DMA,
        pltpu.SemaphoreType.DMA,
    ],
)


def _tc_post_body(p_ref, o_ref):
    x = p_ref[...]
    o_ref[...] = jnp.where(x > 0.0, x, jnp.exp(x) - 1.0)


_tc_post = pl.pallas_call(
    _tc_post_body,
    out_shape=jax.ShapeDtypeStruct((N, F), jnp.float32),
)


def kernel(non_zero, input, W, a):
    src = non_zero[0, :]
    dst = non_zero[1, :]
    a12 = a.reshape(NC, F)
    h, f12 = _tc_pre(input, W, a12)
    hflat = h.reshape(NC * N, FH)
    src2 = src.reshape(NS, NCH, CH)
    dst2 = dst.reshape(NS, NCH, CH)
    zeros = jnp.zeros((NS, ROWS_PER_TILE, FH), jnp.float32)
    ee, hp = _sc_edge(src2, dst2, f12, hflat, zeros)
    out = _tc_post(hp)
    return out, ee.reshape(E)


# speedup vs baseline: 1.4832x; 1.1936x over previous
import jax
import jax.numpy as jnp
from jax import lax
from jax.experimental import pallas as pl
from jax.experimental.pallas import tpu as pltpu
from jax.experimental.pallas import tpu_sc as plsc

N = 10000
E = 320000
F = 128

NC = 2
NS = 16
FH = F // NC
E_PER = E // NS
CH = 80
NCH = E_PER // CH
NBUF = 5
ROWS_PER_TILE = N // NS


def _tc_pre_body(x_ref, w_ref, a12_ref, h_ref, f12_ref):
    h = jnp.dot(x_ref[...], w_ref[...], preferred_element_type=jnp.float32)
    h_ref[...] = h
    f12_ref[...] = lax.dot_general(
        a12_ref[...], h, (((1,), (1,)), ((), ())),
        preferred_element_type=jnp.float32)


_tc_pre = pl.pallas_call(
    _tc_pre_body,
    out_shape=[
        jax.ShapeDtypeStruct((N, F), jnp.float32),
        jax.ShapeDtypeStruct((NC, N), jnp.float32),
    ],
)


def _sc_body(src2_hbm, dst2_hbm, f12_hbm, hflat_hbm, zeros_hbm,
             ee_hbm, hp_hbm,
             src2_v, dst2_v, w2_v, shared,
             gsem_0, gsem_1, gsem_2, gsem_3, gsem_4,
             ssem_0, ssem_1, ssem_2, ssem_3, ssem_4):
    c = lax.axis_index("c")
    s = lax.axis_index("s")

    pltpu.sync_copy(src2_hbm.at[s], src2_v)
    pltpu.sync_copy(dst2_hbm.at[s], dst2_v)

    pltpu.sync_copy(zeros_hbm.at[s],
                    shared.at[pl.ds(s * ROWS_PER_TILE, ROWS_PER_TILE)])

    def wphase(f1_v, f2_v):
        pltpu.sync_copy(f12_hbm.at[0], f1_v)
        pltpu.sync_copy(f12_hbm.at[1], f2_v)

        @plsc.parallel_loop(0, NCH, unroll=2)
        def wbody(ci):
            for k in range(CH // 16):
                si = src2_v[ci, pl.ds(k * 16, 16)]
                di = dst2_v[ci, pl.ds(k * 16, 16)]
                sc = (plsc.load_gather(f1_v, [si])
                      + plsc.load_gather(f2_v, [di]))
                lr = jnp.where(sc >= 0.0, sc, sc * 0.2)
                w2_v[ci, pl.ds(k * 16, 16)] = jnp.exp(-lr)
                dst2_v[ci, pl.ds(k * 16, 16)] = di * 2 + c

    pl.run_scoped(wphase, pltpu.VMEM((N,), jnp.float32),
                  pltpu.VMEM((N,), jnp.float32))

    @pl.when(c == 0)
    def _():
        pltpu.sync_copy(w2_v, ee_hbm.at[s])

    plsc.subcore_barrier()

    def scale(rows, ci):
        civ = jnp.full((16,), ci, jnp.int32)

        @plsc.parallel_loop(0, CH, unroll=8)
        def ebody(k):
            wv = plsc.load_gather(
                w2_v, [civ, jnp.full((16,), k, jnp.int32)])
            for j in range(FH // 16):
                rows[k, pl.ds(j * 16, 16)] = rows[k, pl.ds(j * 16, 16)] * wv

    def mphase(rows_0, rows_1, rows_2, rows_3, rows_4):
        rows = [rows_0, rows_1, rows_2, rows_3, rows_4]
        gsems = [gsem_0, gsem_1, gsem_2, gsem_3, gsem_4]
        ssems = [ssem_0, ssem_1, ssem_2, ssem_3, ssem_4]

        for b in range(NBUF):
            pltpu.async_copy(hflat_hbm.at[dst2_v.at[b]], rows[b], gsems[b])

        def mbody(cio, _):
            ci0 = cio * NBUF
            for b in range(NBUF):
                ci = ci0 + b
                pltpu.make_async_copy(hflat_hbm.at[dst2_v.at[ci]], rows[b],
                                      gsems[b]).wait()
                scale(rows[b], ci)
                pltpu.async_copy(rows[b], shared.at[src2_v.at[ci]], ssems[b],
                                 add=True)
            for b in range(NBUF):
                ci = ci0 + b
                pltpu.make_async_copy(rows[b], shared.at[src2_v.at[ci]],
                                      ssems[b]).wait()

                @pl.when(ci + NBUF < NCH)
                def _():
                    pltpu.async_copy(hflat_hbm.at[dst2_v.at[ci + NBUF]],
                                     rows[b], gsems[b])

            return 0

        lax.fori_loop(0, NCH // NBUF, mbody, 0)

    pl.run_scoped(mphase, *([pltpu.VMEM((CH, FH), jnp.float32)] * NBUF))
    plsc.subcore_barrier()

    pltpu.sync_copy(shared.at[pl.ds(s * ROWS_PER_TILE, ROWS_PER_TILE)],
                    hp_hbm.at[pl.ds(s * ROWS_PER_TILE, ROWS_PER_TILE),
                              pl.ds(c * FH, FH)])


_sc_edge = pl.kernel(
    _sc_body,
    out_type=[
        jax.ShapeDtypeStruct((NS, NCH, CH), jnp.float32),
        jax.ShapeDtypeStruct((N, F), jnp.float32),
    ],
    mesh=plsc.VectorSubcoreMesh(core_axis_name="c", subcore_axis_name="s"),
    compiler_params=pltpu.CompilerParams(
        needs_layout_passes=False, use_tc_tiling_on_sc=False),
    scratch_types=[
        pltpu.VMEM((NCH, CH), jnp.int32),
        pltpu.VMEM((NCH, CH), jnp.int32),
        pltpu.VMEM((NCH, CH), jnp.float32),
    ]
    + [pltpu.VMEM_SHARED((N, FH), jnp.float32)]
    + [pltpu.SemaphoreType.DMA] * 10,
)


def _tc_post_body(p_ref, o_ref):
    x = p_ref[...]
    o_ref[...] = jnp.where(x > 0.0, x, jnp.exp(x) - 1.0)


_tc_post = pl.pallas_call(
    _tc_post_body,
    out_shape=jax.ShapeDtypeStruct((N, F), jnp.float32),
)


def kernel(non_zero, input, W, a):
    src = non_zero[0, :]
    dst = non_zero[1, :]
    a12 = a.reshape(NC, F)
    h, f12 = _tc_pre(input, W, a12)
    hflat = h.reshape(NC * N, FH)
    src2 = src.reshape(NS, NCH, CH)
    dst2 = dst.reshape(NS, NCH, CH)
    zeros = jnp.zeros((NS, ROWS_PER_TILE, FH), jnp.float32)
    ee, hp = _sc_edge(src2, dst2, f12, hflat, zeros)
    out = _tc_post(hp)
    return out, ee.reshape(E)


# speedup vs baseline: 1.8159x; 1.2243x over previous
import jax
import jax.numpy as jnp
from jax import lax
from jax.experimental import pallas as pl
from jax.experimental.pallas import tpu as pltpu
from jax.experimental.pallas import tpu_sc as plsc

N = 10000
E = 320000
F = 128

NC = 2
NS = 16
FH = F // NC
E_PER = E // NS
CH = 80
NCH = E_PER // CH
NBUF = 5
ROWS_PER_TILE = N // NS


def _tc_pre_body(x_ref, w_ref, a12_ref, hb_ref, f12_ref):
    h = jnp.dot(x_ref[...], w_ref[...], preferred_element_type=jnp.float32)
    hb_ref[...] = h.astype(jnp.bfloat16)
    f12_ref[...] = lax.dot_general(
        a12_ref[...], h, (((1,), (1,)), ((), ())),
        preferred_element_type=jnp.float32)


_tc_pre = pl.pallas_call(
    _tc_pre_body,
    out_shape=[
        jax.ShapeDtypeStruct((N, F), jnp.bfloat16),
        jax.ShapeDtypeStruct((NC, N), jnp.float32),
    ],
)


def _sc_body(src2_hbm, dst2_hbm, f12_hbm, hflat_hbm, zeros_hbm,
             ee_hbm, hp_hbm,
             src2_v, dst2_v, w2_v, shared,
             gsem_0, gsem_1, gsem_2, gsem_3, gsem_4,
             ssem_0, ssem_1, ssem_2):
    c = lax.axis_index("c")
    s = lax.axis_index("s")

    pltpu.sync_copy(src2_hbm.at[s], src2_v)
    pltpu.sync_copy(dst2_hbm.at[s], dst2_v)

    pltpu.sync_copy(zeros_hbm.at[s],
                    shared.at[pl.ds(s * ROWS_PER_TILE, ROWS_PER_TILE)])

    def wphase(f1_v, f2_v):
        pltpu.sync_copy(f12_hbm.at[0], f1_v)
        pltpu.sync_copy(f12_hbm.at[1], f2_v)

        @plsc.parallel_loop(0, NCH, unroll=2)
        def wbody(ci):
            for k in range(CH // 16):
                si = src2_v[ci, pl.ds(k * 16, 16)]
                di = dst2_v[ci, pl.ds(k * 16, 16)]
                sc = (plsc.load_gather(f1_v, [si])
                      + plsc.load_gather(f2_v, [di]))
                lr = jnp.where(sc >= 0.0, sc, sc * 0.2)
                w2_v[ci, pl.ds(k * 16, 16)] = jnp.exp(-lr)
                dst2_v[ci, pl.ds(k * 16, 16)] = di * 2 + c

    pl.run_scoped(wphase, pltpu.VMEM((N,), jnp.float32),
                  pltpu.VMEM((N,), jnp.float32))

    @pl.when(c == 0)
    def _():
        pltpu.sync_copy(w2_v, ee_hbm.at[s])

    plsc.subcore_barrier()

    def scale(rbf, rf, ci):
        civ = jnp.full((16,), ci, jnp.int32)

        @plsc.parallel_loop(0, CH, unroll=8)
        def ebody(k):
            wv = plsc.load_gather(
                w2_v, [civ, jnp.full((16,), k, jnp.int32)])
            for j in range(FH // 32):
                raw = rbf[k, pl.ds(j * 32, 32)]
                u0, u1 = plsc.unpack(raw, format=plsc.PackFormat.INTERLEAVED)
                rf[k, pl.ds(j * 32, 16)] = u0 * wv
                rf[k, pl.ds(j * 32 + 16, 16)] = u1 * wv

    def mphase(bf_0, bf_1, bf_2, bf_3, bf_4, rf_0, rf_1, rf_2):
        rbfs = [bf_0, bf_1, bf_2, bf_3, bf_4]
        rfs = [rf_0, rf_1, rf_2]
        gsems = [gsem_0, gsem_1, gsem_2, gsem_3, gsem_4]
        ssems = [ssem_0, ssem_1, ssem_2]
        prev_dist = [2, 2, 5, 3, 3]

        for b in range(NBUF):
            pltpu.async_copy(hflat_hbm.at[dst2_v.at[b]], rbfs[b], gsems[b])

        def mbody(cio, _):
            ci0 = cio * NBUF
            for b in range(NBUF):
                ci = ci0 + b
                rf = rfs[b % 3]
                ssem = ssems[b % 3]
                pltpu.make_async_copy(hflat_hbm.at[dst2_v.at[ci]], rbfs[b],
                                      gsems[b]).wait()

                def _drain():
                    pltpu.make_async_copy(
                        rf, shared.at[src2_v.at[ci - prev_dist[b]]],
                        ssem).wait()

                if b < 3:
                    pl.when(cio > 0)(_drain)
                else:
                    _drain()

                scale(rbfs[b], rf, ci)
                pltpu.async_copy(rf, shared.at[src2_v.at[ci]], ssem,
                                 add=True)

                @pl.when(ci + NBUF < NCH)
                def _():
                    pltpu.async_copy(hflat_hbm.at[dst2_v.at[ci + NBUF]],
                                     rbfs[b], gsems[b])

            return 0

        lax.fori_loop(0, NCH // NBUF, mbody, 0)
        for rf, ssem, ci in ((rf_0, ssem_0, NCH - 2), (rf_1, ssem_1, NCH - 1),
                             (rf_2, ssem_2, NCH - 3)):
            pltpu.make_async_copy(rf, shared.at[src2_v.at[ci]], ssem).wait()

    pl.run_scoped(
        mphase,
        *([pltpu.VMEM((CH, FH), jnp.bfloat16)] * NBUF
          + [pltpu.VMEM((CH, FH), jnp.float32)] * 3))
    plsc.subcore_barrier()

    pltpu.sync_copy(shared.at[pl.ds(s * ROWS_PER_TILE, ROWS_PER_TILE)],
                    hp_hbm.at[pl.ds(s * ROWS_PER_TILE, ROWS_PER_TILE),
                              pl.ds(c * FH, FH)])


_sc_edge = pl.kernel(
    _sc_body,
    out_type=[
        jax.ShapeDtypeStruct((NS, NCH, CH), jnp.float32),
        jax.ShapeDtypeStruct((N, F), jnp.float32),
    ],
    mesh=plsc.VectorSubcoreMesh(core_axis_name="c", subcore_axis_name="s"),
    compiler_params=pltpu.CompilerParams(
        needs_layout_passes=False, use_tc_tiling_on_sc=False),
    scratch_types=[
        pltpu.VMEM((NCH, CH), jnp.int32),
        pltpu.VMEM((NCH, CH), jnp.int32),
        pltpu.VMEM((NCH, CH), jnp.float32),
    ]
    + [pltpu.VMEM_SHARED((N, FH), jnp.float32)]
    + [pltpu.SemaphoreType.DMA] * 8,
)


_PERM = [0] * F
for _c in range(NC):
    for _j in range(FH // 32):
        for _i in range(16):
            _PERM[_c * FH + 32 * _j + _i] = _c * FH + 32 * _j + 2 * _i
            _PERM[_c * FH + 32 * _j + 16 + _i] = _c * FH + 32 * _j + 2 * _i + 1


def _tc_post_body(p_ref, perm_ref, o_ref):
    x = jnp.dot(p_ref[...], perm_ref[...], preferred_element_type=jnp.float32)
    o_ref[...] = jnp.where(x > 0.0, x, jnp.exp(x) - 1.0)


_tc_post = pl.pallas_call(
    _tc_post_body,
    out_shape=jax.ShapeDtypeStruct((N, F), jnp.float32),
)


def kernel(non_zero, input, W, a):
    src = non_zero[0, :]
    dst = non_zero[1, :]
    a12 = a.reshape(NC, F)
    hb, f12 = _tc_pre(input, W, a12)
    hflat = hb.reshape(NC * N, FH)
    src2 = src.reshape(NS, NCH, CH)
    dst2 = dst.reshape(NS, NCH, CH)
    zeros = jnp.zeros((NS, ROWS_PER_TILE, FH), jnp.float32)
    ee, hp = _sc_edge(src2, dst2, f12, hflat, zeros)
    perm_m = jax.nn.one_hot(jnp.array(_PERM, jnp.int32), F, dtype=jnp.float32)
    out = _tc_post(hp, perm_m)
    return out, ee.reshape(E)


# speedup vs baseline: 1.9293x; 1.0624x over previous
import jax
import jax.numpy as jnp
from jax import lax
from jax.experimental import pallas as pl
from jax.experimental.pallas import tpu as pltpu
from jax.experimental.pallas import tpu_sc as plsc

N = 10000
E = 320000
F = 128

NC = 2
NS = 16
FH = F // NC
E_PER = E // NS
CH = 80
NCH = E_PER // CH
NBUF = 5
ROWS_PER_TILE = N // NS


def _tc_pre_body(x_ref, w_ref, a12_ref, hb_ref, f12_ref):
    h = jnp.dot(x_ref[...], w_ref[...], preferred_element_type=jnp.float32)
    hb_ref[...] = h.astype(jnp.bfloat16)
    f12_ref[...] = lax.dot_general(
        a12_ref[...], h, (((1,), (1,)), ((), ())),
        preferred_element_type=jnp.float32)


_tc_pre = pl.pallas_call(
    _tc_pre_body,
    out_shape=[
        jax.ShapeDtypeStruct((N, F), jnp.bfloat16),
        jax.ShapeDtypeStruct((NC, N), jnp.float32),
    ],
)


def _sc_body(nz4_hbm, f12_hbm, hflat_hbm, zeros_hbm,
             ee_hbm, hp_hbm,
             src2_v, dst2_v, w2_v, shared,
             gsem_0, gsem_1, gsem_2, gsem_3, gsem_4,
             ssem_0, ssem_1, ssem_2):
    c = lax.axis_index("c")
    s = lax.axis_index("s")

    pltpu.sync_copy(nz4_hbm.at[0, s], src2_v)
    pltpu.sync_copy(nz4_hbm.at[1, s], dst2_v)

    pltpu.sync_copy(zeros_hbm.at[s],
                    shared.at[pl.ds(s * ROWS_PER_TILE, ROWS_PER_TILE)])

    def wphase(f1_v, f2_v):
        pltpu.sync_copy(f12_hbm.at[0], f1_v)
        pltpu.sync_copy(f12_hbm.at[1], f2_v)

        @plsc.parallel_loop(0, NCH, unroll=2)
        def wbody(ci):
            for k in range(CH // 16):
                si = src2_v[ci, pl.ds(k * 16, 16)]
                di = dst2_v[ci, pl.ds(k * 16, 16)]
                sc = (plsc.load_gather(f1_v, [si])
                      + plsc.load_gather(f2_v, [di]))
                lr = jnp.where(sc >= 0.0, sc, sc * 0.2)
                w2_v[ci, pl.ds(k * 16, 16)] = jnp.exp(-lr)
                dst2_v[ci, pl.ds(k * 16, 16)] = di * 2 + c

    pl.run_scoped(wphase, pltpu.VMEM((N,), jnp.float32),
                  pltpu.VMEM((N,), jnp.float32))

    @pl.when(c == 0)
    def _():
        pltpu.sync_copy(w2_v, ee_hbm.at[s])

    plsc.subcore_barrier()

    def scale(rbf, rf, ci):
        civ = jnp.full((16,), ci, jnp.int32)

        @plsc.parallel_loop(0, CH, unroll=8)
        def ebody(k):
            wv = plsc.load_gather(
                w2_v, [civ, jnp.full((16,), k, jnp.int32)])
            for j in range(FH // 32):
                raw = rbf[k, pl.ds(j * 32, 32)]
                u0, u1 = plsc.unpack(raw, format=plsc.PackFormat.INTERLEAVED)
                rf[k, pl.ds(j * 32, 16)] = u0 * wv
                rf[k, pl.ds(j * 32 + 16, 16)] = u1 * wv

    def mphase(bf_0, bf_1, bf_2, bf_3, bf_4, rf_0, rf_1, rf_2):
        rbfs = [bf_0, bf_1, bf_2, bf_3, bf_4]
        rfs = [rf_0, rf_1, rf_2]
        gsems = [gsem_0, gsem_1, gsem_2, gsem_3, gsem_4]
        ssems = [ssem_0, ssem_1, ssem_2]
        prev_dist = [2, 2, 5, 3, 3]

        for b in range(NBUF):
            pltpu.async_copy(hflat_hbm.at[dst2_v.at[b]], rbfs[b], gsems[b])

        def mbody(cio, _):
            ci0 = cio * NBUF
            for b in range(NBUF):
                ci = ci0 + b
                rf = rfs[b % 3]
                ssem = ssems[b % 3]
                pltpu.make_async_copy(hflat_hbm.at[dst2_v.at[ci]], rbfs[b],
                                      gsems[b]).wait()

                def _drain():
                    pltpu.make_async_copy(
                        rf, shared.at[src2_v.at[ci - prev_dist[b]]],
                        ssem).wait()

                if b < 3:
                    pl.when(cio > 0)(_drain)
                else:
                    _drain()

                scale(rbfs[b], rf, ci)
                pltpu.async_copy(rf, shared.at[src2_v.at[ci]], ssem,
                                 add=True)

                @pl.when(ci + NBUF < NCH)
                def _():
                    pltpu.async_copy(hflat_hbm.at[dst2_v.at[ci + NBUF]],
                                     rbfs[b], gsems[b])

            return 0

        lax.fori_loop(0, NCH // NBUF, mbody, 0)
        for rf, ssem, ci in ((rf_0, ssem_0, NCH - 2), (rf_1, ssem_1, NCH - 1),
                             (rf_2, ssem_2, NCH - 3)):
            pltpu.make_async_copy(rf, shared.at[src2_v.at[ci]], ssem).wait()

    pl.run_scoped(
        mphase,
        *([pltpu.VMEM((CH, FH), jnp.bfloat16)] * NBUF
          + [pltpu.VMEM((CH, FH), jnp.float32)] * 3))
    plsc.subcore_barrier()

    pltpu.sync_copy(shared.at[pl.ds(s * ROWS_PER_TILE, ROWS_PER_TILE)],
                    hp_hbm.at[pl.ds(s * ROWS_PER_TILE, ROWS_PER_TILE),
                              pl.ds(c * FH, FH)])


_sc_edge = pl.kernel(
    _sc_body,
    out_type=[
        jax.ShapeDtypeStruct((NS, NCH, CH), jnp.float32),
        jax.ShapeDtypeStruct((N, F), jnp.float32),
    ],
    mesh=plsc.VectorSubcoreMesh(core_axis_name="c", subcore_axis_name="s"),
    compiler_params=pltpu.CompilerParams(
        needs_layout_passes=False, use_tc_tiling_on_sc=False),
    scratch_types=[
        pltpu.VMEM((NCH, CH), jnp.int32),
        pltpu.VMEM((NCH, CH), jnp.int32),
        pltpu.VMEM((NCH, CH), jnp.float32),
    ]
    + [pltpu.VMEM_SHARED((N, FH), jnp.float32)]
    + [pltpu.SemaphoreType.DMA] * 8,
)


_PERM = [0] * F
for _c in range(NC):
    for _j in range(FH // 32):
        for _i in range(16):
            _PERM[_c * FH + 32 * _j + _i] = _c * FH + 32 * _j + 2 * _i
            _PERM[_c * FH + 32 * _j + 16 + _i] = _c * FH + 32 * _j + 2 * _i + 1


def _tc_post_body(p_ref, perm_ref, o_ref):
    x = jnp.dot(p_ref[...], perm_ref[...], preferred_element_type=jnp.float32)
    o_ref[...] = jnp.where(x > 0.0, x, jnp.exp(x) - 1.0)


_tc_post = pl.pallas_call(
    _tc_post_body,
    out_shape=jax.ShapeDtypeStruct((N, F), jnp.float32),
)


def kernel(non_zero, input, W, a):
    a12 = a.reshape(NC, F)
    hb, f12 = _tc_pre(input, W, a12)
    hflat = hb.reshape(NC * N, FH)
    nz4 = non_zero.reshape(NC, NS, NCH, CH)
    zeros = jnp.zeros((NS, ROWS_PER_TILE, FH), jnp.float32)
    ee, hp = _sc_edge(nz4, f12, hflat, zeros)
    perm_m = jax.nn.one_hot(jnp.array(_PERM, jnp.int32), F, dtype=jnp.float32)
    out = _tc_post(hp, perm_m)
    return out, ee.reshape(E)
